# baseline (speedup 1.0000x reference)
import functools

import jax
import jax.numpy as jnp
from jax import lax
from jax.experimental import pallas as pl
from jax.experimental.pallas import tpu as pltpu
from jax.experimental.pallas import tpu_sc as plsc

N_ATOMS_C = 100000
MAX_NEIGH = 32
CHUNK = 80
N_CHUNKS_TOTAL = N_ATOMS_C // CHUNK
IDX_PER_CHUNK = CHUNK * MAX_NEIGH
GATHER_SPLIT = 128
N_GATHERS = IDX_PER_CHUNK // GATHER_SPLIT
IDX_ROWS = N_ATOMS_C * MAX_NEIGH // GATHER_SPLIT
CUTOFF2 = 36.0
BIG_WORKERS = N_CHUNKS_TOTAL - 32 * (N_CHUNKS_TOTAL // 32)
CHUNKS_SMALL = N_CHUNKS_TOTAL // 32

_mesh = plsc.VectorSubcoreMesh(core_axis_name="c", subcore_axis_name="s")


@functools.partial(
    pl.kernel,
    mesh=_mesh,
    compiler_params=pltpu.CompilerParams(
        use_tc_tiling_on_sc=False, needs_layout_passes=False),
    out_type=(
        jax.ShapeDtypeStruct((N_ATOMS_C, 3), jnp.float32),
        jax.ShapeDtypeStruct((32 * 16,), jnp.float32),
    ),
    scratch_types=[
        pltpu.VMEM((CHUNK, MAX_NEIGH), jnp.int32),
        pltpu.VMEM((CHUNK, MAX_NEIGH), jnp.int32),
        pltpu.VMEM((N_GATHERS, GATHER_SPLIT), jnp.int32),
        pltpu.VMEM((N_GATHERS, GATHER_SPLIT), jnp.int32),
        pltpu.VMEM((IDX_PER_CHUNK, 8), jnp.float32),
        pltpu.VMEM((IDX_PER_CHUNK, 8), jnp.float32),
        pltpu.VMEM((CHUNK, 3), jnp.float32),
        pltpu.VMEM((CHUNK, 3), jnp.float32),
        pltpu.VMEM((CHUNK,), jnp.int32),
        pltpu.VMEM((CHUNK,), jnp.int32),
        pltpu.VMEM((CHUNK, 3), jnp.float32),
        pltpu.VMEM((CHUNK, 3), jnp.float32),
        pltpu.VMEM((16,), jnp.float32),
        pltpu.VMEM((800, 3), jnp.float32),
        pltpu.VMEM((800, 8), jnp.float32),
        pltpu.VMEM_SHARED((102400, 8), jnp.float32),
        pltpu.SemaphoreType.DMA,
        pltpu.SemaphoreType.DMA,
    ],
)
def _lj_sc(pos_hbm, idx_hbm, nn_hbm, fout_hbm, eout_hbm,
           idx_v0, idx_v1, idxw_v0, idxw_v1, rows_v0, rows_v1, own_v0, own_v1,
           nn_v0, nn_v1, fout_v0, fout_v1, e_v, pv3, pv8, pos_sh,
           sem0, sem1):
    cid = lax.axis_index("c")
    sid = lax.axis_index("s")
    wid = sid * 2 + cid
    cnt = jnp.where(wid < BIG_WORKERS, CHUNKS_SMALL + 1, CHUNKS_SMALL)
    start = wid * (CHUNKS_SMALL + 1) - jnp.maximum(wid - BIG_WORKERS, 0)
    lanes = lax.iota(jnp.int32, 16)
    zeros = jnp.zeros((16,), jnp.float32)
    col0 = jnp.zeros((16,), jnp.int32)
    col1 = col0 + 1
    col2 = col0 + 2
    bufs = ((idx_v0, idxw_v0, rows_v0, own_v0, nn_v0, fout_v0, sem0),
            (idx_v1, idxw_v1, rows_v1, own_v1, nn_v1, fout_v1, sem1))

    e_v[...] = zeros
    n_blk = jnp.where(sid < 15, 8, 5)

    def stage_block(blk, _):
        a0 = sid * 6400 + blk * 800
        pltpu.sync_copy(pos_hbm.at[pl.ds(a0, 800)], pv3)
        for g in range(50):
            ar = g * 16 + lanes
            x = plsc.load_gather(pv3, [ar, col0])
            y = plsc.load_gather(pv3, [ar, col1])
            z = plsc.load_gather(pv3, [ar, col2])
            plsc.store_scatter(pv8, [ar, col0], x)
            plsc.store_scatter(pv8, [ar, col1], y)
            plsc.store_scatter(pv8, [ar, col2], z)
        pltpu.sync_copy(pv8, pos_sh.at[pl.ds(a0, 800)])
        return 0

    lax.fori_loop(0, n_blk, stage_block, 0)
    plsc.subcore_barrier()

    def stage(c, b, guarded):
        idx_v, idxw_v, rows_v, own_v, nn_v, _, sem = bufs[b]

        def do():
            ch = start + c
            row0 = ch * CHUNK
            pltpu.sync_copy(idx_hbm.at[pl.ds(row0, CHUNK)], idx_v)
            pltpu.sync_copy(pos_hbm.at[pl.ds(row0, CHUNK)], own_v)
            pltpu.sync_copy(nn_hbm.at[pl.ds(row0, CHUNK)], nn_v)
            for i in range(CHUNK):
                h0 = idx_v[i, pl.ds(0, 16)]
                h1 = idx_v[i, pl.ds(16, 16)]
                f = i * MAX_NEIGH
                idxw_v[f // GATHER_SPLIT, pl.ds(f % GATHER_SPLIT, 16)] = h0
                f += 16
                idxw_v[f // GATHER_SPLIT, pl.ds(f % GATHER_SPLIT, 16)] = h1
            for j in range(N_GATHERS):
                sl = pl.ds(j * GATHER_SPLIT, GATHER_SPLIT)
                pltpu.async_copy(pos_sh.at[idxw_v.at[j]],
                                 rows_v.at[sl], sem)

        if guarded:
            pl.when(c < cnt)(do)
        else:
            do()

    def compute(c, b):
        idx_v, idxw_v, rows_v, own_v, nn_v, fout_v, sem = bufs[b]
        for j in range(N_GATHERS):
            sl = pl.ds(j * GATHER_SPLIT, GATHER_SPLIT)
            pltpu.make_async_copy(pos_sh.at[idxw_v.at[j]],
                                  rows_v.at[sl], sem).wait()

        def i_body(i0, _):
            ai = i0 * 16 + lanes
            xi = plsc.load_gather(own_v, [ai, col0])
            yi = plsc.load_gather(own_v, [ai, col1])
            zi = plsc.load_gather(own_v, [ai, col2])
            nn16 = nn_v[pl.ds(i0 * 16, 16)]
            rbase = ai * MAX_NEIGH
            fx = fy = fz = e = zeros
            for m in range(MAX_NEIGH):
                r = rbase + m
                xj = plsc.load_gather(rows_v, [r, col0])
                yj = plsc.load_gather(rows_v, [r, col1])
                zj = plsc.load_gather(rows_v, [r, col2])
                dx = xj - xi
                dy = yj - yi
                dz = zj - zi
                r2 = dx * dx + dy * dy + dz * dz
                valid = (nn16 > m) & (r2 < CUTOFF2) & (r2 > 1e-12)
                inv = 1.0 / r2
                s6 = inv * inv * inv
                s12 = s6 * s6
                e = e + jnp.where(valid, s12 - s6, 0.0)
                fp = jnp.where(valid, (s12 + s12 - s6) * inv, 0.0)
                fx = fx + fp * dx
                fy = fy + fp * dy
                fz = fz + fp * dz
            plsc.store_scatter(fout_v, [ai, col0], -24.0 * fx)
            plsc.store_scatter(fout_v, [ai, col1], -24.0 * fy)
            plsc.store_scatter(fout_v, [ai, col2], -24.0 * fz)
            e_v[...] = e_v[...] + e
            return 0

        lax.fori_loop(0, CHUNK // 16, i_body, 0)
        pltpu.sync_copy(fout_v,
                        fout_hbm.at[pl.ds((start + c) * CHUNK, CHUNK)])

    stage(0, 0, False)
    stage(1, 1, False)

    def pair_body(k, _):
        c = k * 2
        compute(c, 0)
        stage(c + 2, 0, True)
        compute(c + 1, 1)
        stage(c + 3, 1, True)
        return 0

    n_pairs = (cnt - 1) // 2
    lax.fori_loop(0, n_pairs, pair_body, 0)
    compute(2 * n_pairs, 0)
    pl.when(cnt - 2 * n_pairs == 2)(lambda: compute(2 * n_pairs + 1, 1))

    e_v[...] = 2.0 * e_v[...]
    pltpu.sync_copy(e_v, eout_hbm.at[pl.ds(wid * 16, 16)])


def kernel(positions, neighbor_matrix, num_neighbors):
    forces, eout = _lj_sc(positions.astype(jnp.float32),
                          neighbor_matrix.astype(jnp.int32),
                          num_neighbors.astype(jnp.int32))
    energies = jnp.sum(eout, keepdims=True)
    return energies, forces


# speedup vs baseline: 1.0609x; 1.0609x over previous
import functools

import jax
import jax.numpy as jnp
from jax import lax
from jax.experimental import pallas as pl
from jax.experimental.pallas import tpu as pltpu
from jax.experimental.pallas import tpu_sc as plsc

N_ATOMS_C = 100000
MAX_NEIGH = 32
CHUNK = 80
N_CHUNKS_TOTAL = N_ATOMS_C // CHUNK
IDX_PER_CHUNK = CHUNK * MAX_NEIGH
GATHER_SPLIT = 128
N_GATHERS = IDX_PER_CHUNK // GATHER_SPLIT
CUTOFF2 = 36.0
BIG_WORKERS = N_CHUNKS_TOTAL - 32 * (N_CHUNKS_TOTAL // 32)
CHUNKS_SMALL = N_CHUNKS_TOTAL // 32
SH_PAD = 102400

_mesh = plsc.VectorSubcoreMesh(core_axis_name="c", subcore_axis_name="s")


@functools.partial(
    pl.kernel,
    mesh=_mesh,
    compiler_params=pltpu.CompilerParams(
        use_tc_tiling_on_sc=False, needs_layout_passes=False),
    out_type=(
        jax.ShapeDtypeStruct((3, N_ATOMS_C), jnp.float32),
        jax.ShapeDtypeStruct((32 * 16,), jnp.float32),
    ),
    scratch_types=[
        pltpu.VMEM((CHUNK, MAX_NEIGH), jnp.int32),
        pltpu.VMEM((CHUNK, MAX_NEIGH), jnp.int32),
        pltpu.VMEM((N_GATHERS, GATHER_SPLIT), jnp.int32),
        pltpu.VMEM((N_GATHERS, GATHER_SPLIT), jnp.int32),
        pltpu.VMEM((3, IDX_PER_CHUNK), jnp.float32),
        pltpu.VMEM((3, IDX_PER_CHUNK), jnp.float32),
        pltpu.VMEM((CHUNK, 3), jnp.float32),
        pltpu.VMEM((CHUNK, 3), jnp.float32),
        pltpu.VMEM((CHUNK,), jnp.int32),
        pltpu.VMEM((CHUNK,), jnp.int32),
        pltpu.VMEM((3, CHUNK), jnp.float32),
        pltpu.VMEM((3, CHUNK), jnp.float32),
        pltpu.VMEM((16,), jnp.float32),
        pltpu.VMEM((800, 3), jnp.float32),
        pltpu.VMEM((3, 800), jnp.float32),
        pltpu.VMEM_SHARED((3, SH_PAD), jnp.float32),
        pltpu.SemaphoreType.DMA,
        pltpu.SemaphoreType.DMA,
    ],
)
def _lj_sc(pos_hbm, idx_hbm, nn_hbm, fout_hbm, eout_hbm,
           idx_v0, idx_v1, idxw_v0, idxw_v1, rows_v0, rows_v1, own_v0, own_v1,
           nn_v0, nn_v1, fout_v0, fout_v1, e_v, pv3, pvp, pos_sh,
           sem0, sem1):
    cid = lax.axis_index("c")
    sid = lax.axis_index("s")
    wid = sid * 2 + cid
    cnt = jnp.where(wid < BIG_WORKERS, CHUNKS_SMALL + 1, CHUNKS_SMALL)
    start = wid * (CHUNKS_SMALL + 1) - jnp.maximum(wid - BIG_WORKERS, 0)
    lanes = lax.iota(jnp.int32, 16)
    zeros = jnp.zeros((16,), jnp.float32)
    col0 = jnp.zeros((16,), jnp.int32)
    col1 = col0 + 1
    col2 = col0 + 2
    bufs = ((idx_v0, idxw_v0, rows_v0, own_v0, nn_v0, fout_v0, sem0),
            (idx_v1, idxw_v1, rows_v1, own_v1, nn_v1, fout_v1, sem1))

    e_v[...] = zeros
    n_blk = jnp.where(sid < 15, 8, 5)

    def stage_block(blk, _):
        a0 = sid * 6400 + blk * 800
        pltpu.sync_copy(pos_hbm.at[pl.ds(a0, 800)], pv3)
        for g in range(50):
            ar = g * 16 + lanes
            sl = pl.ds(g * 16, 16)
            pvp[0, sl] = plsc.load_gather(pv3, [ar, col0])
            pvp[1, sl] = plsc.load_gather(pv3, [ar, col1])
            pvp[2, sl] = plsc.load_gather(pv3, [ar, col2])
        for p in range(3):
            pltpu.sync_copy(pvp.at[p],
                            pos_sh.at[p, pl.ds(a0, 800)])
        return 0

    lax.fori_loop(0, n_blk, stage_block, 0)
    plsc.subcore_barrier()

    def stage(c, b, guarded):
        idx_v, idxw_v, rows_v, own_v, nn_v, _, sem = bufs[b]

        def do():
            ch = start + c
            row0 = ch * CHUNK
            pltpu.sync_copy(idx_hbm.at[pl.ds(row0, CHUNK)], idx_v)
            pltpu.sync_copy(pos_hbm.at[pl.ds(row0, CHUNK)], own_v)
            pltpu.sync_copy(nn_hbm.at[pl.ds(row0, CHUNK)], nn_v)
            for m in range(MAX_NEIGH):
                for g in range(CHUNK // 16):
                    ar = g * 16 + lanes
                    f = m * CHUNK + g * 16
                    idxw_v[f // GATHER_SPLIT, pl.ds(f % GATHER_SPLIT, 16)] = (
                        plsc.load_gather(idx_v, [ar, col0 + m]))
            for p in range(3):
                for j in range(N_GATHERS):
                    sl = pl.ds(j * GATHER_SPLIT, GATHER_SPLIT)
                    pltpu.async_copy(pos_sh.at[p].at[idxw_v.at[j]],
                                     rows_v.at[p, sl], sem)

        if guarded:
            pl.when(c < cnt)(do)
        else:
            do()

    def compute(c, b):
        idx_v, idxw_v, rows_v, own_v, nn_v, fout_v, sem = bufs[b]
        for p in range(3):
            for j in range(N_GATHERS):
                sl = pl.ds(j * GATHER_SPLIT, GATHER_SPLIT)
                pltpu.make_async_copy(pos_sh.at[p].at[idxw_v.at[j]],
                                      rows_v.at[p, sl], sem).wait()

        def i_body(i0, _):
            ai = i0 * 16 + lanes
            i16 = i0 * 16
            xi = plsc.load_gather(own_v, [ai, col0])
            yi = plsc.load_gather(own_v, [ai, col1])
            zi = plsc.load_gather(own_v, [ai, col2])
            nn16 = nn_v[pl.ds(i16, 16)]
            fx = fy = fz = e = zeros
            for m in range(MAX_NEIGH):
                sl = pl.ds(m * CHUNK + i16, 16)
                dx = rows_v[0, sl] - xi
                dy = rows_v[1, sl] - yi
                dz = rows_v[2, sl] - zi
                r2 = dx * dx + dy * dy + dz * dz
                valid = (nn16 > m) & (r2 < CUTOFF2) & (r2 > 1e-12)
                inv = 1.0 / r2
                s6 = inv * inv * inv
                s12 = s6 * s6
                e = e + jnp.where(valid, s12 - s6, 0.0)
                fp = jnp.where(valid, (s12 + s12 - s6) * inv, 0.0)
                fx = fx + fp * dx
                fy = fy + fp * dy
                fz = fz + fp * dz
            sl16 = pl.ds(i16, 16)
            fout_v[0, sl16] = -24.0 * fx
            fout_v[1, sl16] = -24.0 * fy
            fout_v[2, sl16] = -24.0 * fz
            e_v[...] = e_v[...] + e
            return 0

        lax.fori_loop(0, CHUNK // 16, i_body, 0)
        row0 = (start + c) * CHUNK
        for p in range(3):
            pltpu.sync_copy(fout_v.at[p],
                            fout_hbm.at[p, pl.ds(row0, CHUNK)])

    stage(0, 0, False)
    stage(1, 1, False)

    def pair_body(k, _):
        c = k * 2
        compute(c, 0)
        stage(c + 2, 0, True)
        compute(c + 1, 1)
        stage(c + 3, 1, True)
        return 0

    n_pairs = (cnt - 1) // 2
    lax.fori_loop(0, n_pairs, pair_body, 0)
    compute(2 * n_pairs, 0)
    pl.when(cnt - 2 * n_pairs == 2)(lambda: compute(2 * n_pairs + 1, 1))

    e_v[...] = 2.0 * e_v[...]
    pltpu.sync_copy(e_v, eout_hbm.at[pl.ds(wid * 16, 16)])


def kernel(positions, neighbor_matrix, num_neighbors):
    fout, eout = _lj_sc(positions.astype(jnp.float32),
                        neighbor_matrix.astype(jnp.int32),
                        num_neighbors.astype(jnp.int32))
    energies = jnp.sum(eout, keepdims=True)
    return energies, fout.T


# speedup vs baseline: 1.1930x; 1.1246x over previous
import functools

import jax
import jax.numpy as jnp
from jax import lax
from jax.experimental import pallas as pl
from jax.experimental.pallas import tpu as pltpu
from jax.experimental.pallas import tpu_sc as plsc

N_ATOMS_C = 100000
MAX_NEIGH = 32
CHUNK = 80
N_CHUNKS_TOTAL = N_ATOMS_C // CHUNK
IDX_PER_CHUNK = CHUNK * MAX_NEIGH
GATHER_SPLIT = 128
N_GATHERS = IDX_PER_CHUNK // GATHER_SPLIT
CUTOFF2 = 36.0
BIG_WORKERS = N_CHUNKS_TOTAL - 32 * (N_CHUNKS_TOTAL // 32)
CHUNKS_SMALL = N_CHUNKS_TOTAL // 32
SH_PAD = 102400

_mesh = plsc.VectorSubcoreMesh(core_axis_name="c", subcore_axis_name="s")


@functools.partial(
    pl.kernel,
    mesh=_mesh,
    compiler_params=pltpu.CompilerParams(
        use_tc_tiling_on_sc=False, needs_layout_passes=False),
    out_type=(
        jax.ShapeDtypeStruct((3, N_ATOMS_C), jnp.float32),
        jax.ShapeDtypeStruct((32 * 16,), jnp.float32),
    ),
    scratch_types=[
        pltpu.VMEM((CHUNK, MAX_NEIGH), jnp.int32),
        pltpu.VMEM((CHUNK, MAX_NEIGH), jnp.int32),
        pltpu.VMEM((IDX_PER_CHUNK,), jnp.int32),
        pltpu.VMEM((IDX_PER_CHUNK,), jnp.int32),
        pltpu.VMEM((3, IDX_PER_CHUNK), jnp.float32),
        pltpu.VMEM((3, IDX_PER_CHUNK), jnp.float32),
        pltpu.VMEM((CHUNK, 3), jnp.float32),
        pltpu.VMEM((CHUNK, 3), jnp.float32),
        pltpu.VMEM((CHUNK,), jnp.int32),
        pltpu.VMEM((CHUNK,), jnp.int32),
        pltpu.VMEM((3, CHUNK), jnp.float32),
        pltpu.VMEM((3, CHUNK), jnp.float32),
        pltpu.VMEM((16,), jnp.float32),
        pltpu.VMEM((800, 3), jnp.float32),
        pltpu.VMEM((3, 800), jnp.float32),
        pltpu.VMEM_SHARED((3, SH_PAD), jnp.float32),
        pltpu.SemaphoreType.DMA,
        pltpu.SemaphoreType.DMA,
    ],
)
def _lj_sc(pos_hbm, idx_hbm, nn_hbm, fout_hbm, eout_hbm,
           idx_v0, idx_v1, idxw_v0, idxw_v1, rows_v0, rows_v1, own_v0, own_v1,
           nn_v0, nn_v1, fout_v0, fout_v1, e_v, pv3, pvp, pos_sh,
           sem0, sem1):
    cid = lax.axis_index("c")
    sid = lax.axis_index("s")
    wid = sid * 2 + cid
    cnt = jnp.where(wid < BIG_WORKERS, CHUNKS_SMALL + 1, CHUNKS_SMALL)
    start = wid * (CHUNKS_SMALL + 1) - jnp.maximum(wid - BIG_WORKERS, 0)
    lanes = lax.iota(jnp.int32, 16)
    zeros = jnp.zeros((16,), jnp.float32)
    col0 = jnp.zeros((16,), jnp.int32)
    col1 = col0 + 1
    col2 = col0 + 2
    bufs = ((idx_v0, idxw_v0, rows_v0, own_v0, nn_v0, fout_v0, sem0),
            (idx_v1, idxw_v1, rows_v1, own_v1, nn_v1, fout_v1, sem1))

    e_v[...] = zeros
    n_blk = jnp.where(sid < 15, 8, 5)

    def stage_block(blk, _):
        a0 = sid * 6400 + blk * 800
        pltpu.sync_copy(pos_hbm.at[pl.ds(a0, 800)], pv3)
        for g in range(50):
            ar = g * 16 + lanes
            sl = pl.ds(g * 16, 16)
            pvp[0, sl] = plsc.load_gather(pv3, [ar, col0])
            pvp[1, sl] = plsc.load_gather(pv3, [ar, col1])
            pvp[2, sl] = plsc.load_gather(pv3, [ar, col2])
        for p in range(3):
            pltpu.sync_copy(pvp.at[p],
                            pos_sh.at[p, pl.ds(a0, 800)])
        return 0

    lax.fori_loop(0, n_blk, stage_block, 0)
    plsc.subcore_barrier()

    def stage(c, b, guarded):
        idx_v, idxw_v, rows_v, own_v, nn_v, _, sem = bufs[b]

        def do():
            ch = start + c
            row0 = ch * CHUNK
            pltpu.sync_copy(idx_hbm.at[pl.ds(row0, CHUNK)], idx_v)
            pltpu.sync_copy(pos_hbm.at[pl.ds(row0, CHUNK)], own_v)
            pltpu.sync_copy(nn_hbm.at[pl.ds(row0, CHUNK)], nn_v)
            for m in range(MAX_NEIGH):
                for g in range(CHUNK // 16):
                    ar = g * 16 + lanes
                    f = m * CHUNK + g * 16
                    idxw_v[pl.ds(f, 16)] = (
                        plsc.load_gather(idx_v, [ar, col0 + m]))
            for p in range(3):
                pltpu.async_copy(pos_sh.at[p].at[idxw_v],
                                 rows_v.at[p], sem)

        if guarded:
            pl.when(c < cnt)(do)
        else:
            do()

    def compute(c, b):
        idx_v, idxw_v, rows_v, own_v, nn_v, fout_v, sem = bufs[b]
        for p in range(3):
            pltpu.make_async_copy(pos_sh.at[p].at[idxw_v],
                                  rows_v.at[p], sem).wait()

        def i_body(i0, _):
            ai = i0 * 16 + lanes
            i16 = i0 * 16
            xi = plsc.load_gather(own_v, [ai, col0])
            yi = plsc.load_gather(own_v, [ai, col1])
            zi = plsc.load_gather(own_v, [ai, col2])
            nn16 = nn_v[pl.ds(i16, 16)]
            fx = fy = fz = e = zeros
            for m in range(MAX_NEIGH):
                sl = pl.ds(m * CHUNK + i16, 16)
                dx = rows_v[0, sl] - xi
                dy = rows_v[1, sl] - yi
                dz = rows_v[2, sl] - zi
                r2 = dx * dx + dy * dy + dz * dz
                valid = (nn16 > m) & (r2 < CUTOFF2) & (r2 > 1e-12)
                inv = 1.0 / r2
                s6 = inv * inv * inv
                s12 = s6 * s6
                e = e + jnp.where(valid, s12 - s6, 0.0)
                fp = jnp.where(valid, (s12 + s12 - s6) * inv, 0.0)
                fx = fx + fp * dx
                fy = fy + fp * dy
                fz = fz + fp * dz
            sl16 = pl.ds(i16, 16)
            fout_v[0, sl16] = -24.0 * fx
            fout_v[1, sl16] = -24.0 * fy
            fout_v[2, sl16] = -24.0 * fz
            e_v[...] = e_v[...] + e
            return 0

        lax.fori_loop(0, CHUNK // 16, i_body, 0)
        row0 = (start + c) * CHUNK
        for p in range(3):
            pltpu.sync_copy(fout_v.at[p],
                            fout_hbm.at[p, pl.ds(row0, CHUNK)])

    stage(0, 0, False)
    stage(1, 1, False)

    def pair_body(k, _):
        c = k * 2
        compute(c, 0)
        stage(c + 2, 0, True)
        compute(c + 1, 1)
        stage(c + 3, 1, True)
        return 0

    n_pairs = (cnt - 1) // 2
    lax.fori_loop(0, n_pairs, pair_body, 0)
    compute(2 * n_pairs, 0)
    pl.when(cnt - 2 * n_pairs == 2)(lambda: compute(2 * n_pairs + 1, 1))

    e_v[...] = 2.0 * e_v[...]
    pltpu.sync_copy(e_v, eout_hbm.at[pl.ds(wid * 16, 16)])


def kernel(positions, neighbor_matrix, num_neighbors):
    fout, eout = _lj_sc(positions.astype(jnp.float32),
                        neighbor_matrix.astype(jnp.int32),
                        num_neighbors.astype(jnp.int32))
    energies = jnp.sum(eout, keepdims=True)
    return energies, fout.T


# speedup vs baseline: 1.3042x; 1.0932x over previous
import functools

import jax
import jax.numpy as jnp
from jax import lax
from jax.experimental import pallas as pl
from jax.experimental.pallas import tpu as pltpu
from jax.experimental.pallas import tpu_sc as plsc

N_ATOMS_C = 100000
MAX_NEIGH = 32
CHUNK = 80
N_CHUNKS_TOTAL = N_ATOMS_C // CHUNK
IDX_PER_CHUNK = CHUNK * MAX_NEIGH
GATHER_SPLIT = 128
N_GATHERS = IDX_PER_CHUNK // GATHER_SPLIT
CUTOFF2 = 36.0
BIG_WORKERS = N_CHUNKS_TOTAL - 32 * (N_CHUNKS_TOTAL // 32)
CHUNKS_SMALL = N_CHUNKS_TOTAL // 32
SH_PAD = 102400

_mesh = plsc.VectorSubcoreMesh(core_axis_name="c", subcore_axis_name="s")


@functools.partial(
    pl.kernel,
    mesh=_mesh,
    compiler_params=pltpu.CompilerParams(
        use_tc_tiling_on_sc=False, needs_layout_passes=False),
    out_type=(
        jax.ShapeDtypeStruct((3, N_ATOMS_C), jnp.float32),
        jax.ShapeDtypeStruct((32 * 16,), jnp.float32),
    ),
    scratch_types=[
        pltpu.VMEM((IDX_PER_CHUNK,), jnp.int32),
        pltpu.VMEM((IDX_PER_CHUNK,), jnp.int32),
        pltpu.VMEM((IDX_PER_CHUNK,), jnp.int32),
        pltpu.VMEM((IDX_PER_CHUNK,), jnp.int32),
        pltpu.VMEM((3, IDX_PER_CHUNK), jnp.float32),
        pltpu.VMEM((3, IDX_PER_CHUNK), jnp.float32),
        pltpu.VMEM((CHUNK * 3,), jnp.float32),
        pltpu.VMEM((CHUNK * 3,), jnp.float32),
        pltpu.VMEM((CHUNK,), jnp.int32),
        pltpu.VMEM((CHUNK,), jnp.int32),
        pltpu.VMEM((3, CHUNK), jnp.float32),
        pltpu.VMEM((3, CHUNK), jnp.float32),
        pltpu.VMEM((16,), jnp.float32),
        pltpu.VMEM((2400,), jnp.float32),
        pltpu.VMEM((3, 800), jnp.float32),
        pltpu.VMEM_SHARED((3, SH_PAD), jnp.float32),
        pltpu.SemaphoreType.DMA,
        pltpu.SemaphoreType.DMA,
    ],
)
def _lj_sc(pos_hbm, idx_hbm, nn_hbm, fout_hbm, eout_hbm,
           idx_v0, idx_v1, idxw_v0, idxw_v1, rows_v0, rows_v1, own_v0, own_v1,
           nn_v0, nn_v1, fout_v0, fout_v1, e_v, pv3, pvp, pos_sh,
           sem0, sem1):
    cid = lax.axis_index("c")
    sid = lax.axis_index("s")
    wid = sid * 2 + cid
    cnt = jnp.where(wid < BIG_WORKERS, CHUNKS_SMALL + 1, CHUNKS_SMALL)
    start = wid * (CHUNKS_SMALL + 1) - jnp.maximum(wid - BIG_WORKERS, 0)
    lanes = lax.iota(jnp.int32, 16)
    zeros = jnp.zeros((16,), jnp.float32)
    col0 = jnp.zeros((16,), jnp.int32)
    col1 = col0 + 1
    col2 = col0 + 2
    bufs = ((idx_v0, idxw_v0, rows_v0, own_v0, nn_v0, fout_v0, sem0),
            (idx_v1, idxw_v1, rows_v1, own_v1, nn_v1, fout_v1, sem1))

    e_v[...] = zeros
    n_blk = jnp.where(sid < 15, 8, 5)

    def stage_block(blk, _):
        a0 = sid * 6400 + blk * 800
        pltpu.sync_copy(pos_hbm.at[pl.ds(a0 * 3, 2400)], pv3)
        for g in range(50):
            ar3 = (g * 16 + lanes) * 3
            sl = pl.ds(g * 16, 16)
            pvp[0, sl] = plsc.load_gather(pv3, [ar3])
            pvp[1, sl] = plsc.load_gather(pv3, [ar3 + 1])
            pvp[2, sl] = plsc.load_gather(pv3, [ar3 + 2])
        for p in range(3):
            pltpu.sync_copy(pvp.at[p],
                            pos_sh.at[p, pl.ds(a0, 800)])
        return 0

    lax.fori_loop(0, n_blk, stage_block, 0)
    plsc.subcore_barrier()

    def stage(c, b, guarded):
        idx_v, idxw_v, rows_v, own_v, nn_v, _, sem = bufs[b]

        def do():
            ch = start + c
            row0 = ch * CHUNK
            pltpu.sync_copy(
                idx_hbm.at[pl.ds(row0 * MAX_NEIGH, IDX_PER_CHUNK)], idx_v)
            pltpu.sync_copy(pos_hbm.at[pl.ds(row0 * 3, CHUNK * 3)], own_v)
            pltpu.sync_copy(nn_hbm.at[pl.ds(row0, CHUNK)], nn_v)
            for m in range(MAX_NEIGH):
                for g in range(CHUNK // 16):
                    ar = g * 16 + lanes
                    f = m * CHUNK + g * 16
                    idxw_v[pl.ds(f, 16)] = (
                        plsc.load_gather(idx_v, [ar * MAX_NEIGH + m]))
            for p in range(3):
                pltpu.async_copy(pos_sh.at[p].at[idxw_v],
                                 rows_v.at[p], sem)

        if guarded:
            pl.when(c < cnt)(do)
        else:
            do()

    def compute(c, b):
        idx_v, idxw_v, rows_v, own_v, nn_v, fout_v, sem = bufs[b]
        for p in range(3):
            pltpu.make_async_copy(pos_sh.at[p].at[idxw_v],
                                  rows_v.at[p], sem).wait()

        def i_body(i0, _):
            i16 = i0 * 16
            ai3 = (i16 + lanes) * 3
            xi = plsc.load_gather(own_v, [ai3])
            yi = plsc.load_gather(own_v, [ai3 + 1])
            zi = plsc.load_gather(own_v, [ai3 + 2])
            nn16 = nn_v[pl.ds(i16, 16)]
            fx = fy = fz = e = zeros
            for m in range(MAX_NEIGH):
                sl = pl.ds(m * CHUNK + i16, 16)
                dx = rows_v[0, sl] - xi
                dy = rows_v[1, sl] - yi
                dz = rows_v[2, sl] - zi
                r2 = dx * dx + dy * dy + dz * dz
                valid = (nn16 > m) & (r2 < CUTOFF2) & (r2 > 1e-12)
                inv = 1.0 / r2
                s6 = inv * inv * inv
                s12 = s6 * s6
                e = e + jnp.where(valid, s12 - s6, 0.0)
                fp = jnp.where(valid, (s12 + s12 - s6) * inv, 0.0)
                fx = fx + fp * dx
                fy = fy + fp * dy
                fz = fz + fp * dz
            sl16 = pl.ds(i16, 16)
            fout_v[0, sl16] = -24.0 * fx
            fout_v[1, sl16] = -24.0 * fy
            fout_v[2, sl16] = -24.0 * fz
            e_v[...] = e_v[...] + e
            return 0

        lax.fori_loop(0, CHUNK // 16, i_body, 0)
        row0 = (start + c) * CHUNK
        for p in range(3):
            pltpu.sync_copy(fout_v.at[p],
                            fout_hbm.at[p, pl.ds(row0, CHUNK)])

    stage(0, 0, False)
    stage(1, 1, False)

    def pair_body(k, _):
        c = k * 2
        compute(c, 0)
        stage(c + 2, 0, True)
        compute(c + 1, 1)
        stage(c + 3, 1, True)
        return 0

    n_pairs = (cnt - 1) // 2
    lax.fori_loop(0, n_pairs, pair_body, 0)
    compute(2 * n_pairs, 0)
    pl.when(cnt - 2 * n_pairs == 2)(lambda: compute(2 * n_pairs + 1, 1))

    e_v[...] = 2.0 * e_v[...]
    pltpu.sync_copy(e_v, eout_hbm.at[pl.ds(wid * 16, 16)])


def kernel(positions, neighbor_matrix, num_neighbors):
    fout, eout = _lj_sc(positions.astype(jnp.float32).reshape(-1),
                        neighbor_matrix.astype(jnp.int32).reshape(-1),
                        num_neighbors.astype(jnp.int32))
    energies = jnp.sum(eout, keepdims=True)
    return energies, fout.T


# speedup vs baseline: 1.4024x; 1.0754x over previous
import functools

import jax
import jax.numpy as jnp
from jax import lax
from jax.experimental import pallas as pl
from jax.experimental.pallas import tpu as pltpu
from jax.experimental.pallas import tpu_sc as plsc

N_ATOMS_C = 100000
MAX_NEIGH = 32
CHUNK = 160
N_CHUNKS_TOTAL = N_ATOMS_C // CHUNK
IDX_PER_CHUNK = CHUNK * MAX_NEIGH
GATHER_SPLIT = 128
N_GATHERS = IDX_PER_CHUNK // GATHER_SPLIT
CUTOFF2 = 36.0
BIG_WORKERS = N_CHUNKS_TOTAL - 32 * (N_CHUNKS_TOTAL // 32)
CHUNKS_SMALL = N_CHUNKS_TOTAL // 32
SH_PAD = 102400

_mesh = plsc.VectorSubcoreMesh(core_axis_name="c", subcore_axis_name="s")


@functools.partial(
    pl.kernel,
    mesh=_mesh,
    compiler_params=pltpu.CompilerParams(
        use_tc_tiling_on_sc=False, needs_layout_passes=False),
    out_type=(
        jax.ShapeDtypeStruct((3, N_ATOMS_C), jnp.float32),
        jax.ShapeDtypeStruct((32 * 16,), jnp.float32),
    ),
    scratch_types=[
        pltpu.VMEM((IDX_PER_CHUNK,), jnp.int32),
        pltpu.VMEM((IDX_PER_CHUNK,), jnp.int32),
        pltpu.VMEM((IDX_PER_CHUNK,), jnp.int32),
        pltpu.VMEM((IDX_PER_CHUNK,), jnp.int32),
        pltpu.VMEM((3, IDX_PER_CHUNK), jnp.float32),
        pltpu.VMEM((3, IDX_PER_CHUNK), jnp.float32),
        pltpu.VMEM((CHUNK * 3,), jnp.float32),
        pltpu.VMEM((CHUNK * 3,), jnp.float32),
        pltpu.VMEM((CHUNK,), jnp.int32),
        pltpu.VMEM((CHUNK,), jnp.int32),
        pltpu.VMEM((3, CHUNK), jnp.float32),
        pltpu.VMEM((3, CHUNK), jnp.float32),
        pltpu.VMEM((16,), jnp.float32),
        pltpu.VMEM((2400,), jnp.float32),
        pltpu.VMEM((3, 800), jnp.float32),
        pltpu.VMEM_SHARED((3, SH_PAD), jnp.float32),
        pltpu.SemaphoreType.DMA,
        pltpu.SemaphoreType.DMA,
    ],
)
def _lj_sc(pos_hbm, idx_hbm, nn_hbm, fout_hbm, eout_hbm,
           idx_v0, idx_v1, idxw_v0, idxw_v1, rows_v0, rows_v1, own_v0, own_v1,
           nn_v0, nn_v1, fout_v0, fout_v1, e_v, pv3, pvp, pos_sh,
           sem0, sem1):
    cid = lax.axis_index("c")
    sid = lax.axis_index("s")
    wid = sid * 2 + cid
    cnt = jnp.where(wid < BIG_WORKERS, CHUNKS_SMALL + 1, CHUNKS_SMALL)
    start = wid * (CHUNKS_SMALL + 1) - jnp.maximum(wid - BIG_WORKERS, 0)
    lanes = lax.iota(jnp.int32, 16)
    zeros = jnp.zeros((16,), jnp.float32)
    col0 = jnp.zeros((16,), jnp.int32)
    col1 = col0 + 1
    col2 = col0 + 2
    bufs = ((idx_v0, idxw_v0, rows_v0, own_v0, nn_v0, fout_v0, sem0),
            (idx_v1, idxw_v1, rows_v1, own_v1, nn_v1, fout_v1, sem1))

    e_v[...] = zeros
    n_blk = jnp.where(sid < 15, 8, 5)

    def stage_block(blk, _):
        a0 = sid * 6400 + blk * 800
        pltpu.sync_copy(pos_hbm.at[pl.ds(a0 * 3, 2400)], pv3)
        for g in range(50):
            ar3 = (g * 16 + lanes) * 3
            sl = pl.ds(g * 16, 16)
            pvp[0, sl] = plsc.load_gather(pv3, [ar3])
            pvp[1, sl] = plsc.load_gather(pv3, [ar3 + 1])
            pvp[2, sl] = plsc.load_gather(pv3, [ar3 + 2])
        for p in range(3):
            pltpu.sync_copy(pvp.at[p],
                            pos_sh.at[p, pl.ds(a0, 800)])
        return 0

    lax.fori_loop(0, n_blk, stage_block, 0)
    plsc.subcore_barrier()

    def stage(c, b, guarded):
        idx_v, idxw_v, rows_v, own_v, nn_v, _, sem = bufs[b]

        def do():
            ch = start + c
            row0 = ch * CHUNK
            pltpu.sync_copy(
                idx_hbm.at[pl.ds(row0 * MAX_NEIGH, IDX_PER_CHUNK)], idx_v)
            pltpu.sync_copy(pos_hbm.at[pl.ds(row0 * 3, CHUNK * 3)], own_v)
            pltpu.sync_copy(nn_hbm.at[pl.ds(row0, CHUNK)], nn_v)
            for m in range(MAX_NEIGH):
                for g in range(CHUNK // 16):
                    ar = g * 16 + lanes
                    f = m * CHUNK + g * 16
                    idxw_v[pl.ds(f, 16)] = (
                        plsc.load_gather(idx_v, [ar * MAX_NEIGH + m]))
            for p in range(3):
                pltpu.async_copy(pos_sh.at[p].at[idxw_v],
                                 rows_v.at[p], sem)

        if guarded:
            pl.when(c < cnt)(do)
        else:
            do()

    def compute(c, b):
        idx_v, idxw_v, rows_v, own_v, nn_v, fout_v, sem = bufs[b]
        for p in range(3):
            pltpu.make_async_copy(pos_sh.at[p].at[idxw_v],
                                  rows_v.at[p], sem).wait()

        def i_body(i0, _):
            i16 = i0 * 16
            ai3 = (i16 + lanes) * 3
            xi = plsc.load_gather(own_v, [ai3])
            yi = plsc.load_gather(own_v, [ai3 + 1])
            zi = plsc.load_gather(own_v, [ai3 + 2])
            nn16 = nn_v[pl.ds(i16, 16)]
            fx = fy = fz = e = zeros
            for m in range(MAX_NEIGH):
                sl = pl.ds(m * CHUNK + i16, 16)
                dx = rows_v[0, sl] - xi
                dy = rows_v[1, sl] - yi
                dz = rows_v[2, sl] - zi
                r2 = dx * dx + dy * dy + dz * dz
                valid = (nn16 > m) & (r2 < CUTOFF2) & (r2 > 1e-12)
                inv = 1.0 / r2
                s6 = inv * inv * inv
                s12 = s6 * s6
                e = e + jnp.where(valid, s12 - s6, 0.0)
                fp = jnp.where(valid, (s12 + s12 - s6) * inv, 0.0)
                fx = fx + fp * dx
                fy = fy + fp * dy
                fz = fz + fp * dz
            sl16 = pl.ds(i16, 16)
            fout_v[0, sl16] = -24.0 * fx
            fout_v[1, sl16] = -24.0 * fy
            fout_v[2, sl16] = -24.0 * fz
            e_v[...] = e_v[...] + e
            return 0

        lax.fori_loop(0, CHUNK // 16, i_body, 0)
        row0 = (start + c) * CHUNK
        for p in range(3):
            pltpu.sync_copy(fout_v.at[p],
                            fout_hbm.at[p, pl.ds(row0, CHUNK)])

    stage(0, 0, False)
    stage(1, 1, False)

    def pair_body(k, _):
        c = k * 2
        compute(c, 0)
        stage(c + 2, 0, True)
        compute(c + 1, 1)
        stage(c + 3, 1, True)
        return 0

    n_pairs = (cnt - 1) // 2
    lax.fori_loop(0, n_pairs, pair_body, 0)
    compute(2 * n_pairs, 0)
    pl.when(cnt - 2 * n_pairs == 2)(lambda: compute(2 * n_pairs + 1, 1))

    e_v[...] = 2.0 * e_v[...]
    pltpu.sync_copy(e_v, eout_hbm.at[pl.ds(wid * 16, 16)])


def kernel(positions, neighbor_matrix, num_neighbors):
    fout, eout = _lj_sc(positions.astype(jnp.float32).reshape(-1),
                        neighbor_matrix.astype(jnp.int32).reshape(-1),
                        num_neighbors.astype(jnp.int32))
    energies = jnp.sum(eout, keepdims=True)
    return energies, fout.T
